# baseline (speedup 1.0000x reference)
import functools

import jax
import jax.numpy as jnp
from jax.experimental import pallas as pl
from jax.experimental.pallas import tpu as pltpu

F32 = jnp.float32
_PREC = jax.lax.Precision.DEFAULT
_EXACT = jax.lax.Precision.HIGHEST

NB_CODE = 1024
NQ = 2
DEPTH = 3
RATE = 3
DILS = [RATE ** d for d in range(DEPTH)][::-1]


def _dot(a, b, prec=_PREC):
    return jax.lax.dot_general(a, b, (((1,), (0,)), ((), ())),
                               precision=prec, preferred_element_type=F32)


def _round8(n):
    return (n + 7) // 8 * 8


def _pad_t(h, lo, hi):
    C = h.shape[1]
    parts = []
    if lo:
        parts.append(jnp.zeros((lo, C), F32))
    parts.append(h)
    if hi:
        parts.append(jnp.zeros((hi, C), F32))
    return jnp.concatenate(parts, axis=0) if len(parts) > 1 else h


def _conv_k(hpad, w_ref, b_ref, T_out, dil=1):
    K = w_ref.shape[0]
    acc = _dot(hpad[0:T_out, :], w_ref[0])
    for k in range(1, K):
        acc = acc + _dot(hpad[k * dil:k * dil + T_out, :], w_ref[k])
    return acc + b_ref[:]


def _resblock_chain(h, rbw_ref, rbb_ref, T):
    for j, dil in enumerate(DILS):
        y = jnp.maximum(h, 0.0)
        yp = _pad_t(y, dil, dil)
        acc = _dot(yp[0:T, :], rbw_ref[j, 0])
        acc = acc + _dot(yp[dil:dil + T, :], rbw_ref[j, 1])
        acc = acc + _dot(yp[2 * dil:2 * dil + T, :], rbw_ref[j, 2])
        y1 = jnp.maximum(acc + rbb_ref[j, 0][None, :], 0.0)
        h = h + (_dot(y1, rbw_ref[j, 3]) + rbb_ref[j, 1][None, :])
    return h


def _c1_body(x_ref, w_ref, b_ref, o_ref, *, K, T_out, post_relu):
    acc = _conv_k(x_ref[0], w_ref, b_ref, T_out)
    if post_relu:
        acc = jnp.maximum(acc, 0.0)
    o_ref[0] = acc


def conv1(x, w, b, *, padding=0, post_relu=False):
    N, T, Ci = x.shape
    O, I, K = w.shape
    T_out = T + 2 * padding - (K - 1)
    wt = jnp.transpose(w, (2, 1, 0))
    b2 = b.reshape(1, O)
    Tp = _round8(T + 2 * padding)
    xp = jnp.pad(x, ((0, 0), (padding, Tp - T - padding), (0, 0)))
    return pl.pallas_call(
        functools.partial(_c1_body, K=K, T_out=T_out, post_relu=post_relu),
        grid=(N,),
        in_specs=[
            pl.BlockSpec((1, Tp, Ci), lambda n: (n, 0, 0)),
            pl.BlockSpec((K, I, O), lambda n: (0, 0, 0)),
            pl.BlockSpec((1, O), lambda n: (0, 0)),
        ],
        out_specs=pl.BlockSpec((1, T_out, O), lambda n: (n, 0, 0)),
        out_shape=jax.ShapeDtypeStruct((N, T_out, O), F32),
    )(xp, wt, b2)


def _enc_blk_body(e_ref, o_ref_in, dw_ref, db_ref, rbw_ref, rbb_ref, out_ref,
                  *, T_out):
    e = e_ref[0]
    o = o_ref_in[0]
    h = _dot(e[0:T_out, :], dw_ref[0])
    h = h + _dot(o[0:T_out, :], dw_ref[1])
    h = h + _dot(e[1:1 + T_out, :], dw_ref[2])
    h = h + _dot(o[1:1 + T_out, :], dw_ref[3])
    h = h + db_ref[:]
    out_ref[0] = _resblock_chain(h, rbw_ref, rbb_ref, T_out)


def enc_block(x, blk):
    N, T, C = x.shape
    T_out = T // 2
    Th = T_out + 2
    Th8 = _round8(Th)
    xp = jnp.pad(x, ((0, 0), (1, 2 * Th8 - T - 1), (0, 0)))
    xph = xp.reshape(N, Th8, 2, C)
    ev, od = xph[:, :, 0, :], xph[:, :, 1, :]
    dw = jnp.transpose(blk['dw'], (2, 1, 0))
    db = blk['db'].reshape(1, C)
    rbw = jnp.stack([jnp.stack([rb['w1'][:, :, 0].T, rb['w1'][:, :, 1].T,
                                rb['w1'][:, :, 2].T, rb['w2'][:, :, 0].T])
                     for rb in blk['res']])
    rbb = jnp.stack([jnp.stack([rb['b1'], rb['b2']]) for rb in blk['res']])
    return pl.pallas_call(
        functools.partial(_enc_blk_body, T_out=T_out),
        grid=(N,),
        in_specs=[
            pl.BlockSpec((1, Th8, C), lambda n: (n, 0, 0)),
            pl.BlockSpec((1, Th8, C), lambda n: (n, 0, 0)),
            pl.BlockSpec((4, C, C), lambda n: (0, 0, 0)),
            pl.BlockSpec((1, C), lambda n: (0, 0)),
            pl.BlockSpec((DEPTH, 4, C, C), lambda n: (0, 0, 0, 0)),
            pl.BlockSpec((DEPTH, 2, C), lambda n: (0, 0, 0)),
        ],
        out_specs=pl.BlockSpec((1, T_out, C), lambda n: (n, 0, 0)),
        out_shape=jax.ShapeDtypeStruct((N, T_out, C), F32),
    )(ev, od, dw, db, rbw, rbb)


def _dec_blk_body(x_ref, rbw_ref, rbb_ref, uw_ref, ub_ref, out_ref, *, T):
    h = _resblock_chain(x_ref[0], rbw_ref, rbb_ref, T)
    hm = _pad_t(h, 1, 0)[0:T, :]
    hp = _pad_t(h, 0, 1)[1:T + 1, :]
    ev = _dot(hm, uw_ref[0]) + _dot(h, uw_ref[1]) + ub_ref[:]
    od = _dot(h, uw_ref[2]) + _dot(hp, uw_ref[3]) + ub_ref[:]
    out_ref[0, :, 0, :] = ev
    out_ref[0, :, 1, :] = od


def dec_block(x, blk):
    N, T, C = x.shape
    rbw = jnp.stack([jnp.stack([rb['w1'][:, :, 0].T, rb['w1'][:, :, 1].T,
                                rb['w1'][:, :, 2].T, rb['w2'][:, :, 0].T])
                     for rb in blk['res']])
    rbb = jnp.stack([jnp.stack([rb['b1'], rb['b2']]) for rb in blk['res']])
    w0 = blk['uw'][:, :, 0].T
    w1 = blk['uw'][:, :, 1].T
    w2 = blk['uw'][:, :, 2].T
    uw = jnp.stack([w0, w1 + w2, w0 + w1, w2])
    ub = blk['ub'].reshape(1, C)
    out = pl.pallas_call(
        functools.partial(_dec_blk_body, T=T),
        grid=(N,),
        in_specs=[
            pl.BlockSpec((1, T, C), lambda n: (n, 0, 0)),
            pl.BlockSpec((DEPTH, 4, C, C), lambda n: (0, 0, 0, 0)),
            pl.BlockSpec((DEPTH, 2, C), lambda n: (0, 0, 0)),
            pl.BlockSpec((4, C, C), lambda n: (0, 0, 0)),
            pl.BlockSpec((1, C), lambda n: (0, 0)),
        ],
        out_specs=pl.BlockSpec((1, T, 2, C), lambda n: (n, 0, 0, 0)),
        out_shape=jax.ShapeDtypeStruct((N, T, 2, C), F32),
    )(x, rbw, rbb, uw, ub)
    return out.reshape(N, 2 * T, C)


def _dec_tail_body(x_ref, mw_ref, mb_ref, fw_ref, fb_ref, out_ref, *, T):
    h = jnp.maximum(_conv_k(x_ref[0], mw_ref, mb_ref, T), 0.0)
    hp = _pad_t(h, 1, 1)
    out_ref[0] = _conv_k(hp, fw_ref, fb_ref, T)


def dec_tail(x, p):
    N, T, C = x.shape
    O = p['fin_w'].shape[0]
    Tp = _round8(T + 2)
    xp = jnp.pad(x, ((0, 0), (1, Tp - T - 1), (0, 0)))
    mw = jnp.transpose(p['mid_w'], (2, 1, 0))
    fw = jnp.transpose(p['fin_w'], (2, 1, 0))
    return pl.pallas_call(
        functools.partial(_dec_tail_body, T=T),
        grid=(N,),
        in_specs=[
            pl.BlockSpec((1, Tp, C), lambda n: (n, 0, 0)),
            pl.BlockSpec((3, C, C), lambda n: (0, 0, 0)),
            pl.BlockSpec((1, C), lambda n: (0, 0)),
            pl.BlockSpec((3, C, O), lambda n: (0, 0, 0)),
            pl.BlockSpec((1, O), lambda n: (0, 0)),
        ],
        out_specs=pl.BlockSpec((1, T, O), lambda n: (n, 0, 0)),
        out_shape=jax.ShapeDtypeStruct((N, T, O), F32),
    )(xp, mw, p['mid_b'].reshape(1, C), fw, p['fin_b'].reshape(1, O))


def _vq_body(xf_ref, cb_ref, q_ref, idx_ref, stats_ref):
    xf = xf_ref[:]
    M, C = xf.shape
    residual = xf
    qout = jnp.zeros_like(xf)
    lane = jax.lax.broadcasted_iota(jnp.int32, (M, NB_CODE), 1)
    for q in range(NQ):
        cb = cb_ref[q]
        rn = jnp.sum(residual * residual, axis=1, keepdims=True)
        cn = jnp.sum(cb * cb, axis=1, keepdims=True)
        cross = jax.lax.dot_general(residual, cb, (((1,), (1,)), ((), ())),
                                    precision=_PREC,
                                    preferred_element_type=F32)
        d = rn - 2.0 * cross + jnp.transpose(cn)
        dmin = jnp.min(d, axis=1, keepdims=True)
        idx = jnp.min(jnp.where(d == dmin, lane, NB_CODE), axis=1,
                      keepdims=True)
        onehot = (lane == idx).astype(F32)
        qv = _dot(onehot, cb, _EXACT)
        diff = residual - qv
        closs = jnp.sum(diff * diff, keepdims=True).reshape(1, 1) / (M * C)
        qout = qout + (residual + (qv - residual))
        residual = diff
        pr = jnp.sum(onehot, axis=0, keepdims=True) / M
        ent = -jnp.sum(pr * jnp.log(pr + 1e-10), axis=1, keepdims=True)
        idx_ref[q] = idx
        stats_ref[q:q + 1, 0:1] = closs
        stats_ref[q:q + 1, 1:2] = jnp.exp(ent)
    q_ref[:] = qout


def residual_vq_pallas(xf, codebooks):
    M, C = xf.shape
    return pl.pallas_call(
        _vq_body,
        in_specs=[pl.BlockSpec(memory_space=pltpu.VMEM),
                  pl.BlockSpec(memory_space=pltpu.VMEM)],
        out_specs=[pl.BlockSpec(memory_space=pltpu.VMEM),
                   pl.BlockSpec(memory_space=pltpu.VMEM),
                   pl.BlockSpec(memory_space=pltpu.VMEM)],
        out_shape=[jax.ShapeDtypeStruct((M, C), F32),
                   jax.ShapeDtypeStruct((NQ, M, 1), jnp.int32),
                   jax.ShapeDtypeStruct((NQ, 2), F32)],
    )(xf, codebooks)


def kernel(x, params):
    x = x.astype(F32)
    N = x.shape[0]
    enc = params['encoder']
    h = conv1(x, enc['in_w'], enc['in_b'], padding=1, post_relu=True)
    for blk in enc['downs']:
        h = enc_block(h, blk)
    x_enc = conv1(h, enc['out_w'], enc['out_b'], padding=1)
    Te, C = x_enc.shape[1], x_enc.shape[2]

    xf = x_enc.reshape(N * Te, C)
    qout, idx, stats = residual_vq_pallas(xf, params['codebooks'])
    x_q = qout.reshape(N, Te, C)

    dec = params['decoder']
    h = conv1(x_q, dec['in_w'], dec['in_b'], padding=1, post_relu=True)
    for blk in dec['ups']:
        h = dec_block(h, blk)
    x_out = dec_tail(h, dec)

    code_idx = jnp.transpose(idx.reshape(NQ, N, Te), (1, 2, 0))
    commit = jnp.sum(stats[:, 0])
    perp = jnp.mean(stats[:, 1])
    return (x_out, code_idx, commit, perp)


# speedup vs baseline: 1.0329x; 1.0329x over previous
import functools

import jax
import jax.numpy as jnp
from jax.experimental import pallas as pl
from jax.experimental.pallas import tpu as pltpu

F32 = jnp.float32
_PREC = jax.lax.Precision.DEFAULT
_EXACT = jax.lax.Precision.HIGHEST

NB_CODE = 1024
NQ = 2
DEPTH = 3
RATE = 3
DILS = [RATE ** d for d in range(DEPTH)][::-1]


def _dot(a, b, prec=_PREC):
    return jax.lax.dot_general(a, b, (((1,), (0,)), ((), ())),
                               precision=prec, preferred_element_type=F32)


def _round8(n):
    return (n + 7) // 8 * 8


def _cm_body(x_ref, w_ref, b_ref, o_ref, *, K, dil, M, pre_relu, post_relu):
    x = x_ref[:]
    if pre_relu:
        x = jnp.maximum(x, 0.0)
    acc = _dot(x[0:M, :], w_ref[0])
    for k in range(1, K):
        acc = acc + _dot(x[k * dil:k * dil + M, :], w_ref[k])
    acc = acc + b_ref[:]
    if post_relu:
        acc = jnp.maximum(acc, 0.0)
    o_ref[:] = acc


def _cm_res_body(x_ref, w_ref, b_ref, r_ref, o_ref, *, K, dil, M,
                 pre_relu, post_relu):
    x = x_ref[:]
    if pre_relu:
        x = jnp.maximum(x, 0.0)
    acc = _dot(x[0:M, :], w_ref[0])
    for k in range(1, K):
        acc = acc + _dot(x[k * dil:k * dil + M, :], w_ref[k])
    acc = acc + b_ref[:]
    if post_relu:
        acc = jnp.maximum(acc, 0.0)
    o_ref[:] = acc + r_ref[:]


def conv1(x, w, b, *, padding=0, dilation=1, pre_relu=False, post_relu=False,
          residual=None):
    N, T, Ci = x.shape
    O, I, K = w.shape
    T_out = T + 2 * padding - dilation * (K - 1)
    wt = jnp.transpose(w, (2, 1, 0))
    b2 = b.reshape(1, O)
    Tp = _round8(T + 2 * padding)
    halo = dilation * (K - 1)
    xp = jnp.pad(x, ((0, 0), (padding, Tp - T - padding), (0, 0)))
    xf = xp.reshape(N * Tp, Ci)
    xf = jnp.pad(xf, ((0, _round8(halo)), (0, 0)))
    M = N * Tp
    args = [xf, wt, b2]
    in_specs = [pl.BlockSpec(memory_space=pltpu.VMEM)] * 3
    if residual is not None:
        rp = jnp.pad(residual, ((0, 0), (0, Tp - T_out), (0, 0)))
        args.append(rp.reshape(N * Tp, O))
        in_specs.append(pl.BlockSpec(memory_space=pltpu.VMEM))
        body = functools.partial(_cm_res_body, K=K, dil=dilation, M=M,
                                 pre_relu=pre_relu, post_relu=post_relu)
    else:
        body = functools.partial(_cm_body, K=K, dil=dilation, M=M,
                                 pre_relu=pre_relu, post_relu=post_relu)
    out = pl.pallas_call(
        body,
        in_specs=in_specs,
        out_specs=pl.BlockSpec(memory_space=pltpu.VMEM),
        out_shape=jax.ShapeDtypeStruct((M, O), F32),
    )(*args)
    return out.reshape(N, Tp, O)[:, :T_out]


def _c2_body(e_ref, o_ref_in, w_ref, b_ref, out_ref, *, M):
    e = e_ref[:]
    o = o_ref_in[:]
    acc = _dot(e[0:M, :], w_ref[0])
    acc = acc + _dot(o[0:M, :], w_ref[1])
    acc = acc + _dot(e[1:1 + M, :], w_ref[2])
    acc = acc + _dot(o[1:1 + M, :], w_ref[3])
    out_ref[:] = acc + b_ref[:]


def conv_s2(x, w, b):
    N, T, Ci = x.shape
    O, I, K = w.shape
    T_out = T // 2
    wt = jnp.transpose(w, (2, 1, 0))
    b2 = b.reshape(1, O)
    Th = T_out + 2
    Th8 = _round8(Th)
    xp = jnp.pad(x, ((0, 0), (1, 2 * Th8 - T - 1), (0, 0)))
    xph = xp.reshape(N, Th8, 2, Ci)
    ev = jnp.pad(xph[:, :, 0, :].reshape(N * Th8, Ci), ((0, 8), (0, 0)))
    od = jnp.pad(xph[:, :, 1, :].reshape(N * Th8, Ci), ((0, 8), (0, 0)))
    M = N * Th8
    out = pl.pallas_call(
        functools.partial(_c2_body, M=M),
        in_specs=[pl.BlockSpec(memory_space=pltpu.VMEM)] * 4,
        out_specs=pl.BlockSpec(memory_space=pltpu.VMEM),
        out_shape=jax.ShapeDtypeStruct((M, O), F32),
    )(ev, od, wt, b2)
    return out.reshape(N, Th8, O)[:, :T_out]


def _vq_body(xf_ref, cb_ref, q_ref, idx_ref, stats_ref):
    xf = xf_ref[:]
    M, C = xf.shape
    residual = xf
    qout = jnp.zeros_like(xf)
    lane = jax.lax.broadcasted_iota(jnp.int32, (M, NB_CODE), 1)
    for q in range(NQ):
        cb = cb_ref[q]
        rn = jnp.sum(residual * residual, axis=1, keepdims=True)
        cn = jnp.sum(cb * cb, axis=1, keepdims=True)
        cross = jax.lax.dot_general(residual, cb, (((1,), (1,)), ((), ())),
                                    precision=_PREC,
                                    preferred_element_type=F32)
        d = rn - 2.0 * cross + jnp.transpose(cn)
        dmin = jnp.min(d, axis=1, keepdims=True)
        idx = jnp.min(jnp.where(d == dmin, lane, NB_CODE), axis=1,
                      keepdims=True)
        onehot = (lane == idx).astype(F32)
        qv = _dot(onehot, cb, _EXACT)
        diff = residual - qv
        closs = jnp.sum(diff * diff, keepdims=True).reshape(1, 1) / (M * C)
        qout = qout + (residual + (qv - residual))
        residual = diff
        pr = jnp.sum(onehot, axis=0, keepdims=True) / M
        ent = -jnp.sum(pr * jnp.log(pr + 1e-10), axis=1, keepdims=True)
        idx_ref[q] = idx
        stats_ref[q:q + 1, 0:1] = closs
        stats_ref[q:q + 1, 1:2] = jnp.exp(ent)
    q_ref[:] = qout


def residual_vq_pallas(xf, codebooks):
    M, C = xf.shape
    return pl.pallas_call(
        _vq_body,
        in_specs=[pl.BlockSpec(memory_space=pltpu.VMEM),
                  pl.BlockSpec(memory_space=pltpu.VMEM)],
        out_specs=[pl.BlockSpec(memory_space=pltpu.VMEM),
                   pl.BlockSpec(memory_space=pltpu.VMEM),
                   pl.BlockSpec(memory_space=pltpu.VMEM)],
        out_shape=[jax.ShapeDtypeStruct((M, C), F32),
                   jax.ShapeDtypeStruct((NQ, M, 1), jnp.int32),
                   jax.ShapeDtypeStruct((NQ, 2), F32)],
    )(xf, codebooks)


def _resblock(h, rb, dil):
    y = conv1(h, rb['w1'], rb['b1'], padding=dil, dilation=dil,
              pre_relu=True, post_relu=True)
    return conv1(y, rb['w2'], rb['b2'], residual=h)


def _encoder(x, p):
    h = conv1(x, p['in_w'], p['in_b'], padding=1, post_relu=True)
    for blk in p['downs']:
        h = conv_s2(h, blk['dw'], blk['db'])
        for rb, dil in zip(blk['res'], DILS):
            h = _resblock(h, rb, dil)
    return conv1(h, p['out_w'], p['out_b'], padding=1)


def _decoder(z, p):
    h = conv1(z, p['in_w'], p['in_b'], padding=1, post_relu=True)
    for blk in p['ups']:
        for rb, dil in zip(blk['res'], DILS):
            h = _resblock(h, rb, dil)
        h = jnp.repeat(h, 2, axis=1)
        h = conv1(h, blk['uw'], blk['ub'], padding=1)
    h = conv1(h, p['mid_w'], p['mid_b'], padding=1, post_relu=True)
    return conv1(h, p['fin_w'], p['fin_b'], padding=1)


def kernel(x, params):
    x = x.astype(F32)
    N = x.shape[0]
    x_enc = _encoder(x, params['encoder'])
    Te, C = x_enc.shape[1], x_enc.shape[2]
    xf = x_enc.reshape(N * Te, C)
    qout, idx, stats = residual_vq_pallas(xf, params['codebooks'])
    x_q = qout.reshape(N, Te, C)
    x_out = _decoder(x_q, params['decoder'])
    code_idx = jnp.transpose(idx.reshape(NQ, N, Te), (1, 2, 0))
    commit = jnp.sum(stats[:, 0])
    perp = jnp.mean(stats[:, 1])
    return (x_out, code_idx, commit, perp)
